# Initial kernel scaffold; baseline (speedup 1.0000x reference)
import jax
import jax.numpy as jnp
from jax.experimental import pallas as pl


def kernel(position_sequence, n_particles_per_example, particle_types, senders, receivers, type_emb, enc_node_W0, enc_node_b0, enc_node_W1, enc_node_b1, enc_edge_W0, enc_edge_b0, enc_edge_W1, enc_edge_b1, pe_W0, pe_b0, pe_W1, pe_b1, pn_W0, pn_b0, pn_W1, pn_b1, dec_W0, dec_b0, dec_W1, dec_b1):
    raise NotImplementedError("write your pallas kernel here")


# speedup vs baseline: 2.6066x; 2.6066x over previous
import functools

import jax
import jax.numpy as jnp
from jax import lax
from jax.experimental import pallas as pl
from jax.experimental.pallas import tpu as pltpu
from jax.experimental.pallas import tpu_sc as plsc

N = 10000
E = 160000
LAT = 128
DIM = 3
SEQ = 6
NTYPES = 9
R = 0.05
BOUNDS = ((0.0, 1.0), (0.0, 1.0), (0.0, 1.0))

NC = 2
NS = 16
NW = NC * NS
CH = 128
CPW = 40
E_PAD = NW * CPW * CH
N_CHUNKS = NW * CPW
N_ACC = 10112
ZPT = N_ACC // NS
OPT = 624

BN = 2000
BEB = 2048

@functools.cache
def _mesh():
    return plsc.VectorSubcoreMesh(
        core_axis_name="c", subcore_axis_name="s", num_cores=NC, num_subcores=NS)


def _ln(x):
    m = jnp.mean(x, axis=-1, keepdims=True)
    xc = x - m
    v = jnp.mean(xc * xc, axis=-1, keepdims=True)
    return xc * lax.rsqrt(v + 1e-5)


def _dot(a, b):
    return jnp.dot(a, b, preferred_element_type=jnp.float32)


def _full(shape):
    return pl.BlockSpec(shape, lambda i: (0, 0))


def _rows(bn, w):
    return pl.BlockSpec((bn, w), lambda i: (i, 0))


def _node_enc(feats, W0, b0, W1, b1, Ws, Wr):
    F = feats.shape[1]

    def body(f_ref, W0_ref, b0_ref, W1_ref, b1_ref, Ws_ref, Wr_ref,
             nl_ref, sp_ref, rp_ref):
        x = f_ref[...]
        h = jnp.maximum(_dot(x, W0_ref[...]) + b0_ref[0:1, :], 0.0)
        nl = _ln(_dot(h, W1_ref[...]) + b1_ref[0:1, :])
        nl_ref[...] = nl
        sp_ref[...] = _dot(nl, Ws_ref[...])
        rp_ref[...] = _dot(nl, Wr_ref[...])

    out = pl.pallas_call(
        body,
        grid=(N // BN,),
        in_specs=[
            _rows(BN, F), _full((F, LAT)), _full((8, LAT)),
            _full((LAT, LAT)), _full((8, LAT)),
            _full((LAT, LAT)), _full((LAT, LAT)),
        ],
        out_specs=[_rows(BN, LAT)] * 3,
        out_shape=[jax.ShapeDtypeStruct((N, LAT), jnp.float32)] * 3,
    )(feats, W0, b0, W1, b1, Ws, Wr)
    return out


def _edge0(gs, gr, gps, gpr, eW0, eb0, eW1, eb1, W0e, b0, W1, b1):
    def body(gs_ref, gr_ref, gps_ref, gpr_ref, eW0_ref, eb0_ref, eW1_ref,
             eb1_ref, W0e_ref, b0_ref, W1_ref, b1_ref, el_ref):
        disp = (gps_ref[...] - gpr_ref[...]) * (1.0 / R)
        ed = jnp.sqrt(jnp.sum(disp * disp, axis=-1, keepdims=True))
        w = eW0_ref[...]
        h0 = (disp[:, 0:1] * w[0:1, :] + disp[:, 1:2] * w[1:2, :]
              + disp[:, 2:3] * w[2:3, :] + ed * w[3:4, :] + eb0_ref[0:1, :])
        el0 = _ln(_dot(jnp.maximum(h0, 0.0), eW1_ref[...]) + eb1_ref[0:1, :])
        h = jnp.maximum(
            _dot(el0, W0e_ref[...]) + gs_ref[...] + gr_ref[...] + b0_ref[0:1, :],
            0.0)
        el_ref[...] = el0 + _ln(_dot(h, W1_ref[...]) + b1_ref[0:1, :])

    return pl.pallas_call(
        body,
        grid=(E_PAD // BEB,),
        in_specs=[
            _rows(BEB, LAT), _rows(BEB, LAT), _rows(BEB, 16), _rows(BEB, 16),
            _full((8, LAT)), _full((8, LAT)), _full((LAT, LAT)), _full((8, LAT)),
            _full((LAT, LAT)), _full((8, LAT)), _full((LAT, LAT)), _full((8, LAT)),
        ],
        out_specs=_rows(BEB, LAT),
        out_shape=jax.ShapeDtypeStruct((E_PAD, LAT), jnp.float32),
    )(gs, gr, gps, gpr, eW0, eb0, eW1, eb1, W0e, b0, W1, b1)


def _edge_step(el, gs, gr, W0e, b0, W1, b1):
    def body(el_ref, gs_ref, gr_ref, W0e_ref, b0_ref, W1_ref, b1_ref, out_ref):
        el_ = el_ref[...]
        h = jnp.maximum(
            _dot(el_, W0e_ref[...]) + gs_ref[...] + gr_ref[...] + b0_ref[0:1, :],
            0.0)
        out_ref[...] = el_ + _ln(_dot(h, W1_ref[...]) + b1_ref[0:1, :])

    return pl.pallas_call(
        body,
        grid=(E_PAD // BEB,),
        in_specs=[
            _rows(BEB, LAT), _rows(BEB, LAT), _rows(BEB, LAT),
            _full((LAT, LAT)), _full((8, LAT)), _full((LAT, LAT)), _full((8, LAT)),
        ],
        out_specs=_rows(BEB, LAT),
        out_shape=jax.ShapeDtypeStruct((E_PAD, LAT), jnp.float32),
    )(el, gs, gr, W0e, b0, W1, b1)


def _node_update(nl, agg0, agg1, Wn, Wa, b0, W1, b1, Ws, Wr):
    def body(nl_ref, a0_ref, a1_ref, Wn_ref, Wa_ref, b0_ref, W1_ref, b1_ref,
             Ws_ref, Wr_ref, out_ref, sp_ref, rp_ref):
        nl_ = nl_ref[...]
        agg = a0_ref[...] + a1_ref[...]
        h = jnp.maximum(
            _dot(nl_, Wn_ref[...]) + _dot(agg, Wa_ref[...]) + b0_ref[0:1, :],
            0.0)
        nn = nl_ + _ln(_dot(h, W1_ref[...]) + b1_ref[0:1, :])
        out_ref[...] = nn
        sp_ref[...] = _dot(nn, Ws_ref[...])
        rp_ref[...] = _dot(nn, Wr_ref[...])

    return pl.pallas_call(
        body,
        grid=(N // BN,),
        in_specs=[
            _rows(BN, LAT), _rows(BN, LAT), _rows(BN, LAT),
            _full((LAT, LAT)), _full((LAT, LAT)), _full((8, LAT)),
            _full((LAT, LAT)), _full((8, LAT)),
            _full((LAT, LAT)), _full((LAT, LAT)),
        ],
        out_specs=[_rows(BN, LAT)] * 3,
        out_shape=[jax.ShapeDtypeStruct((N, LAT), jnp.float32)] * 3,
    )(nl, agg0, agg1, Wn, Wa, b0, W1, b1, Ws, Wr)


def _node_final(nl, agg0, agg1, Wn, Wa, b0, W1, b1, dW0, db0, dW1, db1):
    def body(nl_ref, a0_ref, a1_ref, Wn_ref, Wa_ref, b0_ref, W1_ref, b1_ref,
             dW0_ref, db0_ref, dW1_ref, db1_ref, acc_ref):
        nl_ = nl_ref[...]
        agg = a0_ref[...] + a1_ref[...]
        h = jnp.maximum(
            _dot(nl_, Wn_ref[...]) + _dot(agg, Wa_ref[...]) + b0_ref[0:1, :],
            0.0)
        nn = nl_ + _ln(_dot(h, W1_ref[...]) + b1_ref[0:1, :])
        dh = jnp.maximum(_dot(nn, dW0_ref[...]) + db0_ref[0:1, :], 0.0)
        acc_ref[...] = _dot(dh, dW1_ref[...]) + db1_ref[0:1, :]

    return pl.pallas_call(
        body,
        grid=(N // BN,),
        in_specs=[
            _rows(BN, LAT), _rows(BN, LAT), _rows(BN, LAT),
            _full((LAT, LAT)), _full((LAT, LAT)), _full((8, LAT)),
            _full((LAT, LAT)), _full((8, LAT)),
            _full((LAT, LAT)), _full((8, LAT)), _full((LAT, LAT)), _full((8, LAT)),
        ],
        out_specs=_rows(BN, LAT),
        out_shape=jax.ShapeDtypeStruct((N, LAT), jnp.float32),
    )(nl, agg0, agg1, Wn, Wa, b0, W1, b1, dW0, db0, dW1, db1)


def _sc_gather2(sp, rp, sidx, ridx):
    @functools.partial(
        pl.kernel,
        out_type=[jax.ShapeDtypeStruct((E_PAD, LAT), jnp.float32)] * 2,
        mesh=_mesh(),
        scratch_types=[
            pltpu.VMEM((CPW, CH), jnp.int32),
            pltpu.VMEM((CPW, CH), jnp.int32),
            pltpu.VMEM((CH, LAT), jnp.float32),
            pltpu.VMEM((CH, LAT), jnp.float32),
            pltpu.SemaphoreType.DMA,
            pltpu.SemaphoreType.DMA,
        ],
    )
    def k(sp_hbm, rp_hbm, sidx_hbm, ridx_hbm, gs_hbm, gr_hbm,
          si_v, ri_v, bs_v, br_v, sem_s, sem_r):
        w = lax.axis_index("s") * NC + lax.axis_index("c")
        lo = w * CPW
        pltpu.sync_copy(sidx_hbm.at[pl.ds(lo, CPW)], si_v)
        pltpu.sync_copy(ridx_hbm.at[pl.ds(lo, CPW)], ri_v)

        def body(j, carry):
            cid = lo + j
            cs = pltpu.async_copy(sp_hbm.at[si_v.at[j]], bs_v, sem_s)
            cr = pltpu.async_copy(rp_hbm.at[ri_v.at[j]], br_v, sem_r)
            cs.wait()
            cr.wait()
            pltpu.sync_copy(bs_v, gs_hbm.at[pl.ds(cid * CH, CH)])
            pltpu.sync_copy(br_v, gr_hbm.at[pl.ds(cid * CH, CH)])
            return carry

        lax.fori_loop(0, CPW, body, 0)

    return k(sp, rp, sidx, ridx)


def _sc_gather4(sp, rp, pos16, sidx, ridx):
    @functools.partial(
        pl.kernel,
        out_type=[
            jax.ShapeDtypeStruct((E_PAD, LAT), jnp.float32),
            jax.ShapeDtypeStruct((E_PAD, LAT), jnp.float32),
            jax.ShapeDtypeStruct((E_PAD, 16), jnp.float32),
            jax.ShapeDtypeStruct((E_PAD, 16), jnp.float32),
        ],
        mesh=_mesh(),
        compiler_params=pltpu.CompilerParams(use_tc_tiling_on_sc=False),
        scratch_types=[
            pltpu.VMEM((CPW, CH), jnp.int32),
            pltpu.VMEM((CPW, CH), jnp.int32),
            pltpu.VMEM((CH, LAT), jnp.float32),
            pltpu.VMEM((CH, LAT), jnp.float32),
            pltpu.VMEM((CH, 16), jnp.float32),
            pltpu.VMEM((CH, 16), jnp.float32),
            pltpu.SemaphoreType.DMA,
            pltpu.SemaphoreType.DMA,
        ],
    )
    def k(sp_hbm, rp_hbm, pos_hbm, sidx_hbm, ridx_hbm,
          gs_hbm, gr_hbm, gps_hbm, gpr_hbm,
          si_v, ri_v, bs_v, br_v, ps_v, pr_v, sem_a, sem_b):
        w = lax.axis_index("s") * NC + lax.axis_index("c")
        lo = w * CPW
        pltpu.sync_copy(sidx_hbm.at[pl.ds(lo, CPW)], si_v)
        pltpu.sync_copy(ridx_hbm.at[pl.ds(lo, CPW)], ri_v)

        def body(j, carry):
            cid = lo + j
            c1 = pltpu.async_copy(sp_hbm.at[si_v.at[j]], bs_v, sem_a)
            c2 = pltpu.async_copy(rp_hbm.at[ri_v.at[j]], br_v, sem_b)
            c3 = pltpu.async_copy(pos_hbm.at[si_v.at[j]], ps_v, sem_a)
            c4 = pltpu.async_copy(pos_hbm.at[ri_v.at[j]], pr_v, sem_b)
            c1.wait()
            c2.wait()
            c3.wait()
            c4.wait()
            pltpu.sync_copy(bs_v, gs_hbm.at[pl.ds(cid * CH, CH)])
            pltpu.sync_copy(br_v, gr_hbm.at[pl.ds(cid * CH, CH)])
            pltpu.sync_copy(ps_v, gps_hbm.at[pl.ds(cid * CH, CH)])
            pltpu.sync_copy(pr_v, gpr_hbm.at[pl.ds(cid * CH, CH)])
            return carry

        lax.fori_loop(0, CPW, body, 0)

    return k(sp, rp, pos16, sidx, ridx)


def _sc_scatter(el, scidx, zeros_acc):
    @functools.partial(
        pl.kernel,
        out_type=jax.ShapeDtypeStruct((NC, N, LAT), jnp.float32),
        mesh=_mesh(),
        scratch_types=[
            pltpu.VMEM((CPW, CH), jnp.int32),
            pltpu.VMEM((CH, LAT), jnp.float32),
            pltpu.VMEM_SHARED((N_ACC, LAT), jnp.float32),
        ],
    )
    def k(el_hbm, idx_hbm, z_hbm, agg_hbm, idx_v, buf_v, acc_sh):
        c = lax.axis_index("c")
        s = lax.axis_index("s")
        w = s * NC + c
        lo = w * CPW
        pltpu.sync_copy(z_hbm.at[pl.ds(s * ZPT, ZPT)], acc_sh.at[pl.ds(s * ZPT, ZPT)])
        pltpu.sync_copy(idx_hbm.at[pl.ds(lo, CPW)], idx_v)
        plsc.subcore_barrier()

        def body(j, carry):
            cid = lo + j
            pltpu.sync_copy(el_hbm.at[pl.ds(cid * CH, CH)], buf_v)
            pltpu.sync_copy(buf_v, acc_sh.at[idx_v.at[j]], add=True)
            return carry

        lax.fori_loop(0, CPW, body, 0)
        plsc.subcore_barrier()
        pltpu.sync_copy(acc_sh.at[pl.ds(s * OPT, OPT)],
                        agg_hbm.at[c].at[pl.ds(s * OPT, OPT)])

        @pl.when(s == NS - 1)
        def _tail():
            pltpu.sync_copy(acc_sh.at[pl.ds(NS * OPT, N - NS * OPT)],
                            agg_hbm.at[c].at[pl.ds(NS * OPT, N - NS * OPT)])

    return k(el, scidx, zeros_acc)


def kernel(position_sequence, n_particles_per_example, particle_types, senders,
           receivers, type_emb, enc_node_W0, enc_node_b0, enc_node_W1,
           enc_node_b1, enc_edge_W0, enc_edge_b0, enc_edge_W1, enc_edge_b1,
           pe_W0, pe_b0, pe_W1, pe_b1, pn_W0, pn_b0, pn_W1, pn_b1, dec_W0,
           dec_b0, dec_W1, dec_b1):
    del n_particles_per_example
    f32 = jnp.float32

    pos_last = position_sequence[:, -1]
    pos_prev = position_sequence[:, -2]

    vel = jnp.diff(position_sequence, axis=1).reshape(N, (SEQ - 1) * DIM)
    bnd = jnp.array(BOUNDS, dtype=f32)
    dist = jnp.clip(
        jnp.concatenate([pos_last - bnd[:, 0], bnd[:, 1] - pos_last], axis=1) / R,
        -1.0, 1.0)
    onehot = (particle_types[:, None] == jnp.arange(NTYPES)[None, :]).astype(f32)
    feats = jnp.concatenate([vel, dist, onehot], axis=1)
    feats = jnp.pad(feats, ((0, 0), (0, 2)))

    W0n = jnp.concatenate(
        [enc_node_W0[: (SEQ - 1) * DIM + 2 * DIM],
         type_emb @ enc_node_W0[(SEQ - 1) * DIM + 2 * DIM:]], axis=0)
    W0n = jnp.pad(W0n, ((0, 2), (0, 0)))

    def b8(b):
        return jnp.pad(b.reshape(1, -1), ((0, 7), (0, 0)))

    eW0 = jnp.pad(enc_edge_W0, ((0, 4), (0, 0)))
    dW1 = jnp.pad(dec_W1, ((0, 0), (0, LAT - DIM)))
    db1 = jnp.pad(dec_b1, ((0, LAT - DIM),))

    padg = ((0, E_PAD - E),)
    sidx = jnp.pad(senders, padg).reshape(N_CHUNKS, CH)
    ridx = jnp.pad(receivers, padg).reshape(N_CHUNKS, CH)
    ridx_s = jnp.pad(receivers, padg, constant_values=N).reshape(N_CHUNKS, CH)

    pos16 = jnp.pad(pos_last, ((0, 0), (0, 16 - DIM)))
    zeros_acc = jnp.zeros((N_ACC, LAT), f32)

    nl, sp, rp = _node_enc(feats, W0n, b8(enc_node_b0), enc_node_W1,
                           b8(enc_node_b1), pe_W0[0, LAT:2 * LAT],
                           pe_W0[0, 2 * LAT:])
    gs, gr, gps, gpr = _sc_gather4(sp, rp, pos16, sidx, ridx)
    el = _edge0(gs, gr, gps, gpr, eW0, b8(enc_edge_b0), enc_edge_W1,
                b8(enc_edge_b1), pe_W0[0, :LAT], b8(pe_b0[0]), pe_W1[0],
                b8(pe_b1[0]))
    agg = _sc_scatter(el, ridx_s, zeros_acc)

    for s in range(2):
        nl, sp, rp = _node_update(
            nl, agg[0], agg[1], pn_W0[s, :LAT], pn_W0[s, LAT:], b8(pn_b0[s]),
            pn_W1[s], b8(pn_b1[s]), pe_W0[s + 1, LAT:2 * LAT],
            pe_W0[s + 1, 2 * LAT:])
        gs, gr = _sc_gather2(sp, rp, sidx, ridx)
        el = _edge_step(el, gs, gr, pe_W0[s + 1, :LAT], b8(pe_b0[s + 1]),
                        pe_W1[s + 1], b8(pe_b1[s + 1]))
        agg = _sc_scatter(el, ridx_s, zeros_acc)

    acc = _node_final(nl, agg[0], agg[1], pn_W0[2, :LAT], pn_W0[2, LAT:],
                      b8(pn_b0[2]), pn_W1[2], b8(pn_b1[2]), dec_W0, b8(dec_b0),
                      dW1, b8(db1))

    return pos_last + (pos_last - pos_prev) + acc[:, :DIM]
